# Initial kernel scaffold; baseline (speedup 1.0000x reference)
import jax
import jax.numpy as jnp
from jax.experimental import pallas as pl


def kernel(x, edge_index, edge_weight, W):
    raise NotImplementedError("write your pallas kernel here")


# speedup vs baseline: 2.2895x; 2.2895x over previous
import functools

import jax
import jax.numpy as jnp
from jax import lax
from jax.experimental import pallas as pl
from jax.experimental.pallas import tpu as pltpu
from jax.experimental.pallas import tpu_sc as plsc

N_NODES = 10000
N_PAD = 10240
D = 128
E = 320000
K = 10

NS = 16
C = 128
CHUNKS = (E + NS * C - 1) // (NS * C)
EPT = CHUNKS * C
E_PAD = EPT * NS
ROWS_PER_TILE = N_PAD // NS
WB = 128


def _spmm_body(h_hbm, src_hbm, dst_hbm, w_hbm, out_hbm,
               acc, sidx, didx, wbuf, rows, tbuf):
    wid = lax.axis_index("s")

    def _zrow(r, _):
        z = jnp.zeros((16,), jnp.float32)
        for q in range(8):
            tbuf[r, pl.ds(q * 16, 16)] = z
        return _
    lax.fori_loop(0, WB, _zrow, None)
    for i in range(ROWS_PER_TILE // WB):
        pltpu.sync_copy(tbuf, acc.at[pl.ds(wid * ROWS_PER_TILE + i * WB, WB)])
    plsc.subcore_barrier()

    def _chunk(b, _):
        base = wid * EPT + b * C
        pltpu.sync_copy(src_hbm.at[pl.ds(base, C)], sidx)
        pltpu.sync_copy(dst_hbm.at[pl.ds(base, C)], didx)
        pltpu.sync_copy(w_hbm.at[pl.ds(base, C)], wbuf)
        pltpu.sync_copy(h_hbm.at[sidx], rows)

        def _scale(g, _c):
            wv = wbuf[pl.ds(g * 16, 16)]
            for j in range(16):
                wj = jnp.full((16,), wv[j], jnp.float32)
                r = g * 16 + j
                for q in range(8):
                    sl = pl.ds(q * 16, 16)
                    rows[r, sl] = rows[r, sl] * wj
            return _c
        lax.fori_loop(0, C // 16, _scale, None)

        pltpu.sync_copy(rows, acc.at[didx], add=True)
        return _
    lax.fori_loop(0, CHUNKS, _chunk, None)
    plsc.subcore_barrier()

    for i in range(ROWS_PER_TILE // WB):
        start = wid * ROWS_PER_TILE + i * WB
        pltpu.sync_copy(acc.at[pl.ds(start, WB)], tbuf)
        pltpu.sync_copy(tbuf, out_hbm.at[pl.ds(start, WB)])


_spmm = pl.kernel(
    _spmm_body,
    out_type=jax.ShapeDtypeStruct((N_PAD, D), jnp.float32),
    mesh=plsc.VectorSubcoreMesh(core_axis_name="c", subcore_axis_name="s",
                                num_cores=1),
    scratch_types=[
        pltpu.VMEM_SHARED((N_PAD, D), jnp.float32),
        pltpu.VMEM((C,), jnp.int32),
        pltpu.VMEM((C,), jnp.int32),
        pltpu.VMEM((C,), jnp.float32),
        pltpu.VMEM((C, D), jnp.float32),
        pltpu.VMEM((WB, D), jnp.float32),
    ],
)


def _combine_kernel(*refs):
    hs = refs[:K + 1]
    w_ref = refs[K + 1]
    out_ref = refs[K + 2]
    w = w_ref[...]
    acc = None
    for k in range(K + 1):
        hk = hs[k][...]
        s = jnp.dot(hk, w, preferred_element_type=jnp.float32)
        term = s * hk
        acc = term if acc is None else acc + term
    out_ref[...] = acc


def _combine(hs, W):
    R = 1000
    grid = (N_NODES // R,)
    in_specs = [pl.BlockSpec((R, D), lambda i: (i, 0)) for _ in range(K + 1)]
    in_specs.append(pl.BlockSpec((D, 1), lambda i: (0, 0)))
    return pl.pallas_call(
        _combine_kernel,
        grid=grid,
        in_specs=in_specs,
        out_specs=pl.BlockSpec((R, D), lambda i: (i, 0)),
        out_shape=jax.ShapeDtypeStruct((N_NODES, D), jnp.float32),
    )(*hs, W)


@jax.jit
def kernel(x, edge_index, edge_weight, W):
    src = edge_index[0].astype(jnp.int32)
    dst = edge_index[1].astype(jnp.int32)
    pad = E_PAD - E
    src = jnp.concatenate([src, jnp.zeros((pad,), jnp.int32)])
    dst = jnp.concatenate([dst, jnp.zeros((pad,), jnp.int32)])
    w = jnp.concatenate([edge_weight, jnp.zeros((pad,), jnp.float32)])

    h = jnp.pad(x, ((0, N_PAD - N_NODES), (0, 0)))
    hs = [h]
    for _ in range(K):
        h = _spmm(h, src, dst, w)
        hs.append(h)
    return _combine(hs, W)


# speedup vs baseline: 2.4299x; 1.0613x over previous
import jax
import jax.numpy as jnp
from jax import lax
from jax.experimental import pallas as pl
from jax.experimental.pallas import tpu as pltpu
from jax.experimental.pallas import tpu_sc as plsc

N_NODES = 10000
N_PAD = 10240
D = 128
E = 320000
K = 10

NS = 16
C = 64
JB = 16
NBLK = 20
CHUNKS = JB * NBLK
EPT = CHUNKS * C
E_PAD = EPT * NS
ROWS_PER_TILE = N_PAD // NS
WB = 64
NSLOT = 4


def _spmm_body(h_hbm, src_hbm, dst_hbm, w_hbm, out_hbm,
               acc, sbuf, dbuf, wbuf, rows, gsem, ssem):
    wid = lax.axis_index("s")

    def _zrow(r, _):
        z = jnp.zeros((16,), jnp.float32)
        for q in range(8):
            rows[0, r, pl.ds(q * 16, 16)] = z
        return _
    lax.fori_loop(0, WB, _zrow, None)
    for i in range(ROWS_PER_TILE // WB):
        pltpu.sync_copy(rows.at[0],
                        acc.at[pl.ds(wid * ROWS_PER_TILE + i * WB, WB)])
    plsc.subcore_barrier()

    def _scale(sl, j):
        def body(g, _c):
            wv = wbuf[j, pl.ds(g * 16, 16)]
            for t in range(16):
                wj = jnp.full((16,), wv[t], jnp.float32)
                r = g * 16 + t
                for q in range(8):
                    qs = pl.ds(q * 16, 16)
                    rows[sl, r, qs] = rows[sl, r, qs] * wj
            return _c
        lax.fori_loop(0, C // 16, body, None)

    def _block(b, _):
        row0 = wid * CHUNKS + b * JB
        pltpu.sync_copy(src_hbm.at[pl.ds(row0, JB)], sbuf)
        pltpu.sync_copy(dst_hbm.at[pl.ds(row0, JB)], dbuf)
        pltpu.sync_copy(w_hbm.at[pl.ds(row0, JB)], wbuf)
        gd = [None] * NSLOT
        sd = [None] * NSLOT
        gd[0] = pltpu.async_copy(h_hbm.at[sbuf.at[0]], rows.at[0], gsem[0])
        gd[1] = pltpu.async_copy(h_hbm.at[sbuf.at[1]], rows.at[1], gsem[1])
        for j in range(JB):
            sl = j % NSLOT
            gd[sl].wait()
            _scale(sl, j)
            sd[sl] = pltpu.async_copy(rows.at[sl], acc.at[dbuf.at[j]],
                                      ssem[sl], add=True)
            nj = j + 2
            if nj < JB:
                nsl = nj % NSLOT
                if sd[nsl] is not None:
                    sd[nsl].wait()
                gd[nsl] = pltpu.async_copy(h_hbm.at[sbuf.at[nj]],
                                           rows.at[nsl], gsem[nsl])
        sd[(JB - 2) % NSLOT].wait()
        sd[(JB - 1) % NSLOT].wait()
        return _
    lax.fori_loop(0, NBLK, _block, None)
    plsc.subcore_barrier()

    for i in range(ROWS_PER_TILE // WB):
        start = wid * ROWS_PER_TILE + i * WB
        pltpu.sync_copy(acc.at[pl.ds(start, WB)], rows.at[0])
        pltpu.sync_copy(rows.at[0], out_hbm.at[pl.ds(start, WB)])


_spmm = pl.kernel(
    _spmm_body,
    out_type=jax.ShapeDtypeStruct((N_PAD, D), jnp.float32),
    mesh=plsc.VectorSubcoreMesh(core_axis_name="c", subcore_axis_name="s",
                                num_cores=1),
    scratch_types=[
        pltpu.VMEM_SHARED((N_PAD, D), jnp.float32),
        pltpu.VMEM((JB, C), jnp.int32),
        pltpu.VMEM((JB, C), jnp.int32),
        pltpu.VMEM((JB, C), jnp.float32),
        pltpu.VMEM((NSLOT, C, D), jnp.float32),
        [pltpu.SemaphoreType.DMA] * NSLOT,
        [pltpu.SemaphoreType.DMA] * NSLOT,
    ],
)


def _combine_kernel(*refs):
    hs = refs[:K + 1]
    w_ref = refs[K + 1]
    out_ref = refs[K + 2]
    w = w_ref[...]
    acc = None
    for k in range(K + 1):
        hk = hs[k][...]
        s = jnp.dot(hk, w, preferred_element_type=jnp.float32)
        term = s * hk
        acc = term if acc is None else acc + term
    out_ref[...] = acc


def _combine(hs, W):
    R = 1000
    grid = (N_NODES // R,)
    in_specs = [pl.BlockSpec((R, D), lambda i: (i, 0)) for _ in range(K + 1)]
    in_specs.append(pl.BlockSpec((D, 1), lambda i: (0, 0)))
    return pl.pallas_call(
        _combine_kernel,
        grid=grid,
        in_specs=in_specs,
        out_specs=pl.BlockSpec((R, D), lambda i: (i, 0)),
        out_shape=jax.ShapeDtypeStruct((N_NODES, D), jnp.float32),
    )(*hs, W)


@jax.jit
def kernel(x, edge_index, edge_weight, W):
    src = edge_index[0].astype(jnp.int32)
    dst = edge_index[1].astype(jnp.int32)
    pad = E_PAD - E
    src = jnp.concatenate([src, jnp.zeros((pad,), jnp.int32)]).reshape(
        NS * CHUNKS, C)
    dst = jnp.concatenate([dst, jnp.zeros((pad,), jnp.int32)]).reshape(
        NS * CHUNKS, C)
    w = jnp.concatenate([edge_weight, jnp.zeros((pad,), jnp.float32)]).reshape(
        NS * CHUNKS, C)

    h = jnp.pad(x, ((0, N_PAD - N_NODES), (0, 0)))
    hs = [h]
    for _ in range(K):
        h = _spmm(h, src, dst, w)
        hs.append(h)
    return _combine(hs, W)


# speedup vs baseline: 2.7299x; 1.1235x over previous
import jax
import jax.numpy as jnp
from jax import lax
from jax.experimental import pallas as pl
from jax.experimental.pallas import tpu as pltpu
from jax.experimental.pallas import tpu_sc as plsc

N_NODES = 10000
N_PAD = 10240
D = 128
E = 320000
K = 10

NC = 2
NS = 16
NW = NC * NS
SB = 64
NBLOCKS = N_PAD // SB
BPT = NBLOCKS // NW
C = 64
SC_CH = 2
ECHUNKS = E // C
ESUPER = ECHUNKS // SC_CH
ROWS_PER_TILE = N_PAD // NS
WB = 64
OFF_PAD = 176


def _spmm_body(p0_hbm, p1_hbm, ed_hbm, offs_hbm, out0_hbm, out1_hbm,
               acc, offs_v, hloc, tmp, ebuf, orows, ssem):
    cid = lax.axis_index("c")
    sid = lax.axis_index("s")
    wid = sid * NC + cid

    pltpu.sync_copy(offs_hbm, offs_v)

    def _zrow(r, _):
        z = jnp.zeros((16,), jnp.float32)
        for q in range(8):
            orows[0, r, pl.ds(q * 16, 16)] = z
        return _
    lax.fori_loop(0, WB, _zrow, None)
    for i in range(ROWS_PER_TILE // WB):
        pltpu.sync_copy(orows.at[0],
                        acc.at[pl.ds(sid * ROWS_PER_TILE + i * WB, WB)])
    plsc.subcore_barrier()

    def _sblock(sb, _s):
        blk = wid * BPT + sb
        pltpu.sync_copy(p0_hbm.at[pl.ds(blk * SB, SB)], hloc)
        pltpu.sync_copy(p1_hbm.at[pl.ds(blk * SB, SB)], tmp)

        def _hadd(r, _):
            for q in range(8):
                qs = pl.ds(q * 16, 16)
                hloc[r, qs] = hloc[r, qs] + tmp[r, qs]
            return _
        lax.fori_loop(0, SB, _hadd, None)

        iv = wid * BPT + sb
        e0 = offs_v[pl.ds(iv, 16)][0]
        e1 = offs_v[pl.ds(iv + 1, 16)][0]
        t0 = e0 // (SC_CH * C)
        t1 = (e1 + SC_CH * C - 1) // (SC_CH * C)
        sb_base = blk * SB

        def _super(t, _):
            pltpu.sync_copy(ed_hbm.at[pl.ds(t * 4 * SC_CH, 4 * SC_CH)], ebuf)
            sds = []
            for ci in range(SC_CH):
                def _group(g, _g):
                    sl16 = pl.ds(g * 16, 16)
                    srcv = ebuf[4 * ci, sl16]
                    wv = lax.bitcast_convert_type(
                        ebuf[4 * ci + 2, sl16], jnp.float32)
                    gpos = lax.iota(jnp.int32, 16) + (
                        t * (SC_CH * C) + ci * C + g * 16)
                    m = (gpos >= e0) & (gpos < e1)
                    wv = jnp.where(m, wv, jnp.zeros((16,), jnp.float32))
                    for j in range(16):
                        ls = srcv[j] - sb_base
                        ls = jnp.minimum(jnp.maximum(ls, 0), SB - 1)
                        wj = jnp.full((16,), wv[j], jnp.float32)
                        er = g * 16 + j
                        for q in range(8):
                            qs = pl.ds(q * 16, 16)
                            orows[ci, er, qs] = hloc[ls, qs] * wj
                    return _g
                lax.fori_loop(0, C // 16, _group, None)
                sds.append(pltpu.async_copy(
                    orows.at[ci], acc.at[ebuf.at[4 * ci + 1]], ssem[ci],
                    add=True))
            for sd in sds:
                sd.wait()
            return _
        lax.fori_loop(t0, t1, _super, None)
        return _s
    lax.fori_loop(0, BPT, _sblock, None)
    plsc.subcore_barrier()

    for i in range(ROWS_PER_TILE // WB):
        start = sid * ROWS_PER_TILE + i * WB
        pltpu.sync_copy(acc.at[pl.ds(start, WB)], orows.at[0])

        @pl.when(cid == 0)
        def _():
            pltpu.sync_copy(orows.at[0], out0_hbm.at[pl.ds(start, WB)])

        @pl.when(cid == 1)
        def _():
            pltpu.sync_copy(orows.at[0], out1_hbm.at[pl.ds(start, WB)])


_spmm = pl.kernel(
    _spmm_body,
    out_type=(jax.ShapeDtypeStruct((N_PAD, D), jnp.float32),
              jax.ShapeDtypeStruct((N_PAD, D), jnp.float32)),
    mesh=plsc.VectorSubcoreMesh(core_axis_name="c", subcore_axis_name="s",
                                num_cores=NC),
    scratch_types=[
        pltpu.VMEM_SHARED((N_PAD, D), jnp.float32),
        pltpu.VMEM((OFF_PAD,), jnp.int32),
        pltpu.VMEM((SB, D), jnp.float32),
        pltpu.VMEM((SB, D), jnp.float32),
        pltpu.VMEM((4 * SC_CH, C), jnp.int32),
        pltpu.VMEM((SC_CH, C, D), jnp.float32),
        [pltpu.SemaphoreType.DMA] * SC_CH,
    ],
)


def _combine_kernel(*refs):
    x_ref = refs[0]
    w_ref = refs[2 * K + 1]
    out_ref = refs[2 * K + 2]
    w = w_ref[...]
    h0 = x_ref[...]
    acc = jnp.dot(h0, w, preferred_element_type=jnp.float32) * h0
    for k in range(K):
        hk = refs[1 + 2 * k][...] + refs[2 + 2 * k][...]
        s = jnp.dot(hk, w, preferred_element_type=jnp.float32)
        acc = acc + s * hk
    out_ref[...] = acc


def _combine(x_pad, partials, W):
    R = 1000
    grid = (N_NODES // R,)
    n_h = 1 + 2 * K
    in_specs = [pl.BlockSpec((R, D), lambda i: (i, 0)) for _ in range(n_h)]
    in_specs.append(pl.BlockSpec((D, 1), lambda i: (0, 0)))
    flat = [x_pad]
    for p in partials:
        flat.extend([p[0], p[1]])
    return pl.pallas_call(
        _combine_kernel,
        grid=grid,
        in_specs=in_specs,
        out_specs=pl.BlockSpec((R, D), lambda i: (i, 0)),
        out_shape=jax.ShapeDtypeStruct((N_NODES, D), jnp.float32),
    )(*flat, W)


@jax.jit
def kernel(x, edge_index, edge_weight, W):
    src = edge_index[0].astype(jnp.int32)
    dst = edge_index[1].astype(jnp.int32)

    order = jnp.argsort(src)
    ss = src[order]
    dd = dst[order]
    ww = edge_weight[order]
    ed = jnp.stack(
        [ss.reshape(ECHUNKS, C), dd.reshape(ECHUNKS, C),
         lax.bitcast_convert_type(ww, jnp.int32).reshape(ECHUNKS, C),
         jnp.zeros((ECHUNKS, C), jnp.int32)],
        axis=1).reshape(4 * ECHUNKS, C)
    bounds = jnp.arange(0, NBLOCKS + 1, dtype=jnp.int32) * SB
    offs = jnp.searchsorted(ss, bounds).astype(jnp.int32)
    offs = jnp.concatenate(
        [offs, jnp.full((OFF_PAD - NBLOCKS - 1,), E, jnp.int32)])

    x_pad = jnp.pad(x, ((0, N_PAD - N_NODES), (0, 0)))
    zeros = jnp.zeros_like(x_pad)
    partials = []
    p0, p1 = x_pad, zeros
    for _ in range(K):
        p = _spmm(p0, p1, ed, offs)
        partials.append(p)
        p0, p1 = p[0], p[1]
    return _combine(x_pad, partials, W)


# speedup vs baseline: 2.8172x; 1.0320x over previous
import jax
import jax.numpy as jnp
from jax import lax
from jax.experimental import pallas as pl
from jax.experimental.pallas import tpu as pltpu
from jax.experimental.pallas import tpu_sc as plsc

N_NODES = 10000
N_PAD = 10240
D = 128
E = 320000
K = 10

NC = 2
NS = 16
NW = NC * NS
SB = 64
NBLOCKS = N_PAD // SB
BPT = NBLOCKS // NW
C = 64
NCH = 4
PIT = NCH * C
ECHUNKS = E // C
NSUPER = E // (2 * C)
ED_ROWS = 8 * NSUPER + 32
ROWS_PER_TILE = N_PAD // NS
WB = 64
OFF_PAD = 176


def _spmm_body(p0_hbm, p1_hbm, ed_hbm, offs_hbm, out0_hbm, out1_hbm,
               acc, offs_v, hloc, ebuf, orows, didx, esem, ssem):
    cid = lax.axis_index("c")
    sid = lax.axis_index("s")
    wid = sid * NC + cid

    pltpu.sync_copy(offs_hbm, offs_v)

    def _zrow(r, _):
        z = jnp.zeros((16,), jnp.float32)
        for q in range(8):
            orows[0, r, pl.ds(q * 16, 16)] = z
        return _
    lax.fori_loop(0, WB, _zrow, None)
    for i in range(ROWS_PER_TILE // WB):
        pltpu.sync_copy(orows.at[0],
                        acc.at[pl.ds(sid * ROWS_PER_TILE + i * WB, WB)])
    plsc.subcore_barrier()

    def _chunk(bk, su, ci, sl, e0, e1, sb_base):
        for g4 in range(4):
            s16 = pl.ds(g4 * 16, 16)
            didx[sl, s16] = ebuf[bk, 4 * ci + 1, s16]

        def _group(g, _g):
            sl16 = pl.ds(g * 16, 16)
            srcv = ebuf[bk, 4 * ci, sl16]
            wv = lax.bitcast_convert_type(ebuf[bk, 4 * ci + 2, sl16],
                                          jnp.float32)
            gpos = lax.iota(jnp.int32, 16) + ((su * 2 + ci) * C + g * 16)
            m = (gpos >= e0) & (gpos < e1)
            wv = jnp.where(m, wv, jnp.zeros((16,), jnp.float32))
            for j in range(16):
                ls = srcv[j] - sb_base
                ls = jnp.minimum(jnp.maximum(ls, 0), SB - 1)
                wj = jnp.full((16,), wv[j], jnp.float32)
                er = g * 16 + j
                for q in range(8):
                    qs = pl.ds(q * 16, 16)
                    orows[sl, er, qs] = hloc[ls, qs] * wj
            return _g
        lax.fori_loop(0, C // 16, _group, None)

    def _sblock(sb, _s):
        blk = wid * BPT + sb
        pltpu.sync_copy(p0_hbm.at[pl.ds(blk * SB, SB)], hloc)
        pltpu.sync_copy(p1_hbm.at[pl.ds(blk * SB, SB)], orows.at[0])

        def _hadd(r, _):
            for q in range(8):
                qs = pl.ds(q * 16, 16)
                hloc[r, qs] = hloc[r, qs] + orows[0, r, qs]
            return _
        lax.fori_loop(0, SB, _hadd, None)

        iv = wid * BPT + sb
        e0 = offs_v[pl.ds(iv, 16)][0]
        e1 = offs_v[pl.ds(iv + 1, 16)][0]
        p0i = e0 // PIT
        p1i = (e1 + PIT - 1) // PIT
        sb_base = blk * SB

        pltpu.async_copy(ed_hbm.at[pl.ds((2 * p0i) * 8, 8)],
                         ebuf.at[0], esem[0])
        pltpu.async_copy(ed_hbm.at[pl.ds((2 * p0i + 1) * 8, 8)],
                         ebuf.at[1], esem[1])

        def _pipe(p, _):
            sds = [None] * NCH
            for bk in range(2):
                su = 2 * p + bk
                pltpu.make_async_copy(ed_hbm.at[pl.ds(su * 8, 8)],
                                      ebuf.at[bk], esem[bk]).wait()
                for ci in range(2):
                    sl = 2 * bk + ci
                    _chunk(bk, su, ci, sl, e0, e1, sb_base)
                    sds[sl] = pltpu.async_copy(
                        orows.at[sl], acc.at[didx.at[sl]], ssem[sl],
                        add=True)
                pltpu.async_copy(ed_hbm.at[pl.ds((su + 2) * 8, 8)],
                                 ebuf.at[bk], esem[bk])
            for sd in sds:
                sd.wait()
            return _
        lax.fori_loop(p0i, p1i, _pipe, None)

        pltpu.make_async_copy(ed_hbm.at[pl.ds(0, 8)], ebuf.at[0],
                              esem[0]).wait()
        pltpu.make_async_copy(ed_hbm.at[pl.ds(0, 8)], ebuf.at[1],
                              esem[1]).wait()
        return _s
    lax.fori_loop(0, BPT, _sblock, None)
    plsc.subcore_barrier()

    for i in range(ROWS_PER_TILE // WB):
        start = sid * ROWS_PER_TILE + i * WB
        pltpu.sync_copy(acc.at[pl.ds(start, WB)], orows.at[0])

        @pl.when(cid == 0)
        def _():
            pltpu.sync_copy(orows.at[0], out0_hbm.at[pl.ds(start, WB)])

        @pl.when(cid == 1)
        def _():
            pltpu.sync_copy(orows.at[0], out1_hbm.at[pl.ds(start, WB)])


_spmm = pl.kernel(
    _spmm_body,
    out_type=(jax.ShapeDtypeStruct((N_PAD, D), jnp.float32),
              jax.ShapeDtypeStruct((N_PAD, D), jnp.float32)),
    mesh=plsc.VectorSubcoreMesh(core_axis_name="c", subcore_axis_name="s",
                                num_cores=NC),
    scratch_types=[
        pltpu.VMEM_SHARED((N_PAD, D), jnp.float32),
        pltpu.VMEM((OFF_PAD,), jnp.int32),
        pltpu.VMEM((SB, D), jnp.float32),
        pltpu.VMEM((2, 8, C), jnp.int32),
        pltpu.VMEM((NCH, C, D), jnp.float32),
        pltpu.VMEM((NCH, C), jnp.int32),
        [pltpu.SemaphoreType.DMA] * 2,
        [pltpu.SemaphoreType.DMA] * NCH,
    ],
)


def _combine_kernel(*refs):
    x_ref = refs[0]
    w_ref = refs[2 * K + 1]
    out_ref = refs[2 * K + 2]
    w = w_ref[...]
    h0 = x_ref[...]
    acc = jnp.dot(h0, w, preferred_element_type=jnp.float32) * h0
    for k in range(K):
        hk = refs[1 + 2 * k][...] + refs[2 + 2 * k][...]
        s = jnp.dot(hk, w, preferred_element_type=jnp.float32)
        acc = acc + s * hk
    out_ref[...] = acc


def _combine(x_pad, partials, W):
    R = 1000
    grid = (N_NODES // R,)
    n_h = 1 + 2 * K
    in_specs = [pl.BlockSpec((R, D), lambda i: (i, 0)) for _ in range(n_h)]
    in_specs.append(pl.BlockSpec((D, 1), lambda i: (0, 0)))
    flat = [x_pad]
    for p in partials:
        flat.extend([p[0], p[1]])
    return pl.pallas_call(
        _combine_kernel,
        grid=grid,
        in_specs=in_specs,
        out_specs=pl.BlockSpec((R, D), lambda i: (i, 0)),
        out_shape=jax.ShapeDtypeStruct((N_NODES, D), jnp.float32),
    )(*flat, W)


@jax.jit
def kernel(x, edge_index, edge_weight, W):
    src = edge_index[0].astype(jnp.int32)
    dst = edge_index[1].astype(jnp.int32)

    order = jnp.argsort(src)
    ss = src[order]
    dd = dst[order]
    ww = edge_weight[order]
    ed = jnp.stack(
        [ss.reshape(ECHUNKS, C), dd.reshape(ECHUNKS, C),
         lax.bitcast_convert_type(ww, jnp.int32).reshape(ECHUNKS, C),
         jnp.zeros((ECHUNKS, C), jnp.int32)],
        axis=1).reshape(4 * ECHUNKS, C)
    ed = jnp.pad(ed, ((0, ED_ROWS - 4 * ECHUNKS), (0, 0)))
    bounds = jnp.arange(0, NBLOCKS + 1, dtype=jnp.int32) * SB
    offs = jnp.searchsorted(ss, bounds).astype(jnp.int32)
    offs = jnp.concatenate(
        [offs, jnp.full((OFF_PAD - NBLOCKS - 1,), E, jnp.int32)])

    x_pad = jnp.pad(x, ((0, N_PAD - N_NODES), (0, 0)))
    zeros = jnp.zeros_like(x_pad)
    partials = []
    p0, p1 = x_pad, zeros
    for _ in range(K):
        p = _spmm(p0, p1, ed, offs)
        partials.append(p)
        p0, p1 = p[0], p[1]
    return _combine(x_pad, partials, W)


# speedup vs baseline: 2.8342x; 1.0060x over previous
import jax
import jax.numpy as jnp
from jax import lax
from jax.experimental import pallas as pl
from jax.experimental.pallas import tpu as pltpu
from jax.experimental.pallas import tpu_sc as plsc

N_NODES = 10000
N_PAD = 10240
D = 128
E = 320000
K = 10

NC = 2
NS = 16
NW = NC * NS
SB = 64
NBLOCKS = N_PAD // SB
BPT = NBLOCKS // NW
C = 64
NCH = 4
PIT = NCH * C
ECHUNKS = E // C
NSUPER = E // (2 * C)
ED_ROWS = 8 * NSUPER + 32
ROWS_PER_TILE = N_PAD // NS
WB = 64
OFF_PAD = 176


_GDN = lax.GatherDimensionNumbers(offset_dims=(), collapsed_slice_dims=(0,),
                                  start_index_map=(0,))


def _dyn_pick(v, idx):
    return lax.gather(v, idx[:, None], dimension_numbers=_GDN,
                      slice_sizes=(1,),
                      mode=lax.GatherScatterMode.PROMISE_IN_BOUNDS)

def _spmm_body(p0_hbm, p1_hbm, ed_hbm, offs_hbm, out0_hbm, out1_hbm,
               acc, offs_v, hflat, ebuf, orows, didx, esem, ssem):
    cid = lax.axis_index("c")
    sid = lax.axis_index("s")
    wid = sid * NC + cid

    pltpu.sync_copy(offs_hbm, offs_v)

    def _zrow(r, _):
        z = jnp.zeros((16,), jnp.float32)
        for q in range(8):
            orows[0, r, pl.ds(q * 16, 16)] = z
        return _
    lax.fori_loop(0, WB, _zrow, None)
    for i in range(ROWS_PER_TILE // WB):
        pltpu.sync_copy(orows.at[0],
                        acc.at[pl.ds(sid * ROWS_PER_TILE + i * WB, WB)])
    plsc.subcore_barrier()

    def _chunk(bk, su, ci, sl, e0, e1, sb_base):
        for g4 in range(4):
            s16 = pl.ds(g4 * 16, 16)
            didx[sl, s16] = ebuf[bk, 4 * ci + 1, s16]

        def _group(g, _g):
            sl16 = pl.ds(g * 16, 16)
            srcv = ebuf[bk, 4 * ci, sl16]
            wv = lax.bitcast_convert_type(ebuf[bk, 4 * ci + 2, sl16],
                                          jnp.float32)
            gpos = lax.iota(jnp.int32, 16) + ((su * 2 + ci) * C + g * 16)
            m = (gpos >= e0) & (gpos < e1)
            wv = jnp.where(m, wv, jnp.zeros((16,), jnp.float32))
            lsv = jnp.minimum(jnp.maximum(srcv - sb_base, 0), SB - 1)
            ls128 = lsv * D
            for j in range(16):
                jv = jnp.full((16,), j, jnp.int32)
                wj = _dyn_pick(wv, jv)
                base = ls128[j]
                er = g * 16 + j
                for q in range(8):
                    vals = hflat[pl.ds(base + q * 16, 16)]
                    orows[sl, er, pl.ds(q * 16, 16)] = vals * wj
            return _g
        lax.fori_loop(0, C // 16, _group, None)

    def _sblock(sb, _s):
        blk = wid * BPT + sb
        pltpu.sync_copy(p0_hbm.at[pl.ds(blk * SB, SB)], orows.at[0])
        pltpu.sync_copy(p1_hbm.at[pl.ds(blk * SB, SB)], orows.at[1])

        def _hadd(r, _):
            for q in range(8):
                qs = pl.ds(q * 16, 16)
                hflat[pl.ds(r * D + q * 16, 16)] = (
                    orows[0, r, qs] + orows[1, r, qs])
            return _
        lax.fori_loop(0, SB, _hadd, None)

        iv = wid * BPT + sb
        e0 = offs_v[pl.ds(iv, 16)][0]
        e1 = offs_v[pl.ds(iv + 1, 16)][0]
        p0i = e0 // PIT
        p1i = (e1 + PIT - 1) // PIT
        sb_base = blk * SB

        pltpu.async_copy(ed_hbm.at[pl.ds((2 * p0i) * 8, 8)],
                         ebuf.at[0], esem[0])
        pltpu.async_copy(ed_hbm.at[pl.ds((2 * p0i + 1) * 8, 8)],
                         ebuf.at[1], esem[1])

        def _pipe(p, _):
            sds = [None] * NCH
            for bk in range(2):
                su = 2 * p + bk
                pltpu.make_async_copy(ed_hbm.at[pl.ds(su * 8, 8)],
                                      ebuf.at[bk], esem[bk]).wait()
                for ci in range(2):
                    sl = 2 * bk + ci
                    _chunk(bk, su, ci, sl, e0, e1, sb_base)
                    sds[sl] = pltpu.async_copy(
                        orows.at[sl], acc.at[didx.at[sl]], ssem[sl],
                        add=True)
                pltpu.async_copy(ed_hbm.at[pl.ds((su + 2) * 8, 8)],
                                 ebuf.at[bk], esem[bk])
            for sd in sds:
                sd.wait()
            return _
        lax.fori_loop(p0i, p1i, _pipe, None)

        pltpu.make_async_copy(ed_hbm.at[pl.ds(0, 8)], ebuf.at[0],
                              esem[0]).wait()
        pltpu.make_async_copy(ed_hbm.at[pl.ds(0, 8)], ebuf.at[1],
                              esem[1]).wait()
        return _s
    lax.fori_loop(0, BPT, _sblock, None)
    plsc.subcore_barrier()

    for i in range(ROWS_PER_TILE // WB):
        start = sid * ROWS_PER_TILE + i * WB
        pltpu.sync_copy(acc.at[pl.ds(start, WB)], orows.at[0])

        @pl.when(cid == 0)
        def _():
            pltpu.sync_copy(orows.at[0], out0_hbm.at[pl.ds(start, WB)])

        @pl.when(cid == 1)
        def _():
            pltpu.sync_copy(orows.at[0], out1_hbm.at[pl.ds(start, WB)])


_spmm = pl.kernel(
    _spmm_body,
    out_type=(jax.ShapeDtypeStruct((N_PAD, D), jnp.float32),
              jax.ShapeDtypeStruct((N_PAD, D), jnp.float32)),
    mesh=plsc.VectorSubcoreMesh(core_axis_name="c", subcore_axis_name="s",
                                num_cores=NC),
    scratch_types=[
        pltpu.VMEM_SHARED((N_PAD, D), jnp.float32),
        pltpu.VMEM((OFF_PAD,), jnp.int32),
        pltpu.VMEM((SB * D,), jnp.float32),
        pltpu.VMEM((2, 8, C), jnp.int32),
        pltpu.VMEM((NCH, C, D), jnp.float32),
        pltpu.VMEM((NCH, C), jnp.int32),
        [pltpu.SemaphoreType.DMA] * 2,
        [pltpu.SemaphoreType.DMA] * NCH,
    ],
)


def _combine_kernel(*refs):
    x_ref = refs[0]
    w_ref = refs[2 * K + 1]
    out_ref = refs[2 * K + 2]
    w = w_ref[...]
    h0 = x_ref[...]
    acc = jnp.dot(h0, w, preferred_element_type=jnp.float32) * h0
    for k in range(K):
        hk = refs[1 + 2 * k][...] + refs[2 + 2 * k][...]
        s = jnp.dot(hk, w, preferred_element_type=jnp.float32)
        acc = acc + s * hk
    out_ref[...] = acc


def _combine(x_pad, partials, W):
    R = 1000
    grid = (N_NODES // R,)
    n_h = 1 + 2 * K
    in_specs = [pl.BlockSpec((R, D), lambda i: (i, 0)) for _ in range(n_h)]
    in_specs.append(pl.BlockSpec((D, 1), lambda i: (0, 0)))
    flat = [x_pad]
    for p in partials:
        flat.extend([p[0], p[1]])
    return pl.pallas_call(
        _combine_kernel,
        grid=grid,
        in_specs=in_specs,
        out_specs=pl.BlockSpec((R, D), lambda i: (i, 0)),
        out_shape=jax.ShapeDtypeStruct((N_NODES, D), jnp.float32),
    )(*flat, W)


@jax.jit
def kernel(x, edge_index, edge_weight, W):
    src = edge_index[0].astype(jnp.int32)
    dst = edge_index[1].astype(jnp.int32)

    order = jnp.argsort(src)
    ss = src[order]
    dd = dst[order]
    ww = edge_weight[order]
    ed = jnp.stack(
        [ss.reshape(ECHUNKS, C), dd.reshape(ECHUNKS, C),
         lax.bitcast_convert_type(ww, jnp.int32).reshape(ECHUNKS, C),
         jnp.zeros((ECHUNKS, C), jnp.int32)],
        axis=1).reshape(4 * ECHUNKS, C)
    ed = jnp.pad(ed, ((0, ED_ROWS - 4 * ECHUNKS), (0, 0)))
    bounds = jnp.arange(0, NBLOCKS + 1, dtype=jnp.int32) * SB
    offs = jnp.searchsorted(ss, bounds).astype(jnp.int32)
    offs = jnp.concatenate(
        [offs, jnp.full((OFF_PAD - NBLOCKS - 1,), E, jnp.int32)])

    x_pad = jnp.pad(x, ((0, N_PAD - N_NODES), (0, 0)))
    zeros = jnp.zeros_like(x_pad)
    partials = []
    p0, p1 = x_pad, zeros
    for _ in range(K):
        p = _spmm(p0, p1, ed, offs)
        partials.append(p)
        p0, p1 = p[0], p[1]
    return _combine(x_pad, partials, W)
